# baseline (speedup 1.0000x reference)
import dataclasses

import jax
import jax.numpy as jnp
from jax.experimental import pallas as pl
from jax.experimental.pallas import tpu as pltpu
from jax.experimental.pallas import tpu_sc as plsc

N = 10000
E = 320000
D = 128
DE = 16

NC = 2
NS = 16
LANES = 16
K = 80
EPW = E // (NC * NS)
CHUNKS = EPW // K
NP = 10240
RPS = NP // NS
NQ = 1280
QPS = NQ // NS

_HIGHEST = jax.lax.Precision.HIGHEST


def _node_body(x_ref, wn_ref, bn_ref, wm_ref, h_ref, hm_ref):
    h = jnp.maximum(
        jnp.dot(x_ref[...], wn_ref[...], precision=_HIGHEST,
                preferred_element_type=jnp.float32) + bn_ref[...], 0.0)
    h_ref[...] = h
    hm_ref[...] = jnp.dot(h, wm_ref[...], precision=_HIGHEST,
                          preferred_element_type=jnp.float32)


def _node_embed(x, W_node, b_node, W_msg):
    return pl.pallas_call(
        _node_body,
        out_shape=(jax.ShapeDtypeStruct((N, D), jnp.float32),
                   jax.ShapeDtypeStruct((N, D), jnp.float32)),
    )(x, W_node, b_node.reshape(1, D), W_msg)


BE2 = 2000

def _edge_body(ea_ref, w1_ref, b1_ref, w2_ref, b2_ref, eme_ref, emo_ref):
    z = jnp.dot(ea_ref[...], w1_ref[...], preferred_element_type=jnp.float32)
    e = jnp.maximum(z + b1_ref[...], 0.0).astype(jnp.bfloat16)
    em = jnp.dot(e, w2_ref[...],
                 preferred_element_type=jnp.float32) + b2_ref[...]
    eme_ref[...] = em[:, :D]
    emo_ref[...] = em[:, D:]


def _edge_embed(edge_attr, W_edge, b_edge, W_msg, b_msg):
    z2 = jnp.zeros((DE, D), jnp.float32)
    w1 = jnp.block([[W_edge, z2], [z2, W_edge]]).astype(jnp.bfloat16)
    zd = jnp.zeros((D, D), jnp.float32)
    w2 = jnp.block([[W_msg, zd], [zd, W_msg]]).astype(jnp.bfloat16)
    b1 = jnp.concatenate([b_edge, b_edge]).reshape(1, 2 * D)
    b2 = jnp.concatenate([b_msg, b_msg]).reshape(1, 2 * D)
    ea2 = edge_attr.astype(jnp.bfloat16).reshape(E // 2, 2 * DE)
    return pl.pallas_call(
        _edge_body,
        grid=(E // 2 // BE2,),
        in_specs=[
            pl.BlockSpec((BE2, 2 * DE), lambda i: (i, 0)),
            pl.BlockSpec((2 * DE, 2 * D), lambda i: (0, 0)),
            pl.BlockSpec((1, 2 * D), lambda i: (0, 0)),
            pl.BlockSpec((2 * D, 2 * D), lambda i: (0, 0)),
            pl.BlockSpec((1, 2 * D), lambda i: (0, 0)),
        ],
        out_specs=(pl.BlockSpec((BE2, D), lambda i: (i, 0)),
                   pl.BlockSpec((BE2, D), lambda i: (i, 0))),
        out_shape=(jax.ShapeDtypeStruct((E // 2, D), jnp.float32),
                   jax.ShapeDtypeStruct((E // 2, D), jnp.float32)),
    )(ea2, w1, b1, w2, b2)


def _sc_body(hm_hbm, eme_hbm, emo_hbm, src_hbm, dst_hbm, acc_hbm, aux_hbm,
             src_v, dst_v, dstq_v, eme_v, emo_v, g_v, ones_v,
             acc_sh, aux_sh, sem, sem2, sem3, sem4, sem5):
    c = jax.lax.axis_index("c")
    s = jax.lax.axis_index("s")
    wid = c * NS + s

    @pl.loop(0, K)
    def _(r):
        @pl.loop(0, D, step=LANES)
        def _(j):
            g_v[r, pl.ds(j, LANES)] = jnp.zeros((LANES,), jnp.float32)
            ones_v[r, pl.ds(j, LANES)] = jnp.zeros((LANES,), jnp.float32)

    @pl.loop(0, RPS // K)
    def _(q):
        pltpu.sync_copy(g_v, acc_sh.at[pl.ds(s * RPS + q * K, K)])
    pltpu.sync_copy(g_v, aux_sh.at[pl.ds(s * QPS, QPS)])
    plsc.subcore_barrier()

    @pl.loop(0, CHUNKS)
    def _(t):
        base = wid * EPW + t * K
        cp_src = pltpu.async_copy(src_hbm.at[pl.ds(base, K)], src_v, sem2)
        cp_dst = pltpu.async_copy(dst_hbm.at[pl.ds(base, K)], dst_v, sem3)
        base2 = pl.multiple_of(wid * (EPW // 2) + t * (K // 2), 8)
        with jax.named_scope("ph_issue"):
            cp_eme = pltpu.async_copy(eme_hbm.at[pl.ds(base2, K // 2)], eme_v,
                                      sem4)
            cp_emo = pltpu.async_copy(emo_hbm.at[pl.ds(base2, K // 2)], emo_v,
                                      sem5)
            cp_src.wait()
            cp_gather = pltpu.async_copy(hm_hbm.at[src_v], g_v, sem)
            cp_dst.wait()

        with jax.named_scope("ph_ones"):
            @pl.loop(0, K, step=LANES)
            def _(r16):
                sl = pl.ds(r16, LANES)
                d16 = dst_v[sl]
                dstq_v[sl] = jax.lax.shift_right_logical(d16, 3)
                rows = jax.lax.iota(jnp.int32, LANES) + r16
                cols = (d16 & 7) * LANES
                plsc.store_scatter(ones_v, [rows, cols],
                                   jnp.full((LANES,), 1.0, jnp.float32))

        with jax.named_scope("ph_emwait"):
            cp_eme.wait()
            cp_emo.wait()
        with jax.named_scope("ph_gwait"):
            cp_gather.wait()

        with jax.named_scope("ph_compute"):
            @pl.loop(0, K // 2)
            def _(q):
                for h, em_buf in ((0, eme_v), (1, emo_v)):
                    for j in range(0, D, LANES):
                        sl = pl.ds(j, LANES)
                        g_v[2 * q + h, sl] = jnp.maximum(
                            g_v[2 * q + h, sl] + em_buf[q, sl], 0.0)

        with jax.named_scope("ph_scat"):
            cp_acc = pltpu.async_copy(g_v, acc_sh.at[dst_v], sem2, add=True)
            pltpu.sync_copy(ones_v, aux_sh.at[dstq_v], add=True)
            cp_acc.wait()

        with jax.named_scope("ph_clear"):
            @pl.loop(0, K, step=LANES)
            def _(r16):
                sl = pl.ds(r16, LANES)
                rows = jax.lax.iota(jnp.int32, LANES) + r16
                cols = (dst_v[sl] & 7) * LANES
                plsc.store_scatter(ones_v, [rows, cols],
                                   jnp.zeros((LANES,), jnp.float32))

    plsc.subcore_barrier()

    pltpu.sync_copy(acc_sh.at[pl.ds(s * RPS, RPS)],
                    acc_hbm.at[c, pl.ds(s * RPS, RPS)])
    pltpu.sync_copy(aux_sh.at[pl.ds(s * QPS, QPS)],
                    aux_hbm.at[c, pl.ds(s * QPS, QPS)])


def _sc_aggregate(hm, eme, emo, src, dst):
    mesh = plsc.VectorSubcoreMesh(core_axis_name="c", subcore_axis_name="s")
    cp = pltpu.CompilerParams()
    if "needs_layout_passes" in pltpu.CompilerParams.__dataclass_fields__:
        cp = dataclasses.replace(cp, needs_layout_passes=False)
    kern = pl.kernel(
        _sc_body,
        compiler_params=cp,
        out_type=(jax.ShapeDtypeStruct((NC, NP, D), jnp.float32),
                  jax.ShapeDtypeStruct((NC, NQ, D), jnp.float32)),
        mesh=mesh,
        scratch_types=[
            pltpu.VMEM((K,), jnp.int32),
            pltpu.VMEM((K,), jnp.int32),
            pltpu.VMEM((K,), jnp.int32),
            pltpu.VMEM((K // 2, D), jnp.float32),
            pltpu.VMEM((K // 2, D), jnp.float32),
            pltpu.VMEM((K, D), jnp.float32),
            pltpu.VMEM((K, D), jnp.float32),
            pltpu.VMEM_SHARED((NP, D), jnp.float32),
            pltpu.VMEM_SHARED((NQ, D), jnp.float32),
            pltpu.SemaphoreType.DMA,
            pltpu.SemaphoreType.DMA,
            pltpu.SemaphoreType.DMA,
            pltpu.SemaphoreType.DMA,
            pltpu.SemaphoreType.DMA,
        ],
    )
    return kern(hm, eme, emo, src, dst)


BN = 2000

def _upd_body(h_ref, acc_ref, cnt_ref, w1_ref, w2_ref, w3_ref, bu_ref, o_ref):
    agg = acc_ref[0] + acc_ref[1]
    cnt = cnt_ref[...]
    mean = agg / jnp.maximum(cnt, 1.0)
    o = (jnp.dot(h_ref[...], w1_ref[...], precision=_HIGHEST,
                 preferred_element_type=jnp.float32)
         + jnp.dot(agg, w2_ref[...], precision=_HIGHEST,
                   preferred_element_type=jnp.float32)
         + jnp.dot(mean, w3_ref[...], precision=_HIGHEST,
                   preferred_element_type=jnp.float32)
         + bu_ref[...])
    o_ref[...] = jnp.maximum(o, 0.0)


def _node_update(h, acc, cnt, W_upd, b_upd):
    return pl.pallas_call(
        _upd_body,
        grid=(N // BN,),
        in_specs=[
            pl.BlockSpec((BN, D), lambda i: (i, 0)),
            pl.BlockSpec((2, BN, D), lambda i: (0, i, 0)),
            pl.BlockSpec((BN, 1), lambda i: (i, 0)),
            pl.BlockSpec((D, D), lambda i: (0, 0)),
            pl.BlockSpec((D, D), lambda i: (0, 0)),
            pl.BlockSpec((D, D), lambda i: (0, 0)),
            pl.BlockSpec((1, D), lambda i: (0, 0)),
        ],
        out_specs=pl.BlockSpec((BN, D), lambda i: (i, 0)),
        out_shape=jax.ShapeDtypeStruct((N, D), jnp.float32),
    )(h, acc, cnt, W_upd[0:D], W_upd[D:2 * D], W_upd[2 * D:3 * D],
      b_upd.reshape(1, D))


def kernel(x, edge_index, edge_attr, W_node, b_node, W_edge, b_edge,
           W_msg, b_msg, W_upd, b_upd):
    src = edge_index[0].astype(jnp.int32)
    dst = edge_index[1].astype(jnp.int32)
    h, hm = _node_embed(x, W_node, b_node, W_msg)
    eme, emo = _edge_embed(edge_attr, W_edge, b_edge, W_msg, b_msg)
    acc, aux = _sc_aggregate(hm, eme, emo, src, dst)
    auxs = aux[0] + aux[1]
    cnt = auxs.reshape(NQ, 8, LANES)[:, :, 0].reshape(NQ * 8, 1)[:N]
    return _node_update(h, acc, cnt, W_upd, b_upd)


# speedup vs baseline: 1.4505x; 1.4505x over previous
import dataclasses

import jax
import jax.numpy as jnp
from jax.experimental import pallas as pl
from jax.experimental.pallas import tpu as pltpu
from jax.experimental.pallas import tpu_sc as plsc

N = 10000
E = 320000
D = 128
DE = 16

NC = 2
NS = 16
LANES = 16
K = 80
EPW = E // (NC * NS)
CHUNKS = EPW // K
NP = 10240
RPS = NP // NS
NQ = 1280
QPS = NQ // NS

_HIGHEST = jax.lax.Precision.HIGHEST


def _node_body(x_ref, wn_ref, bn_ref, wm_ref, h_ref, hm_ref):
    h = jnp.maximum(
        jnp.dot(x_ref[...], wn_ref[...], precision=_HIGHEST,
                preferred_element_type=jnp.float32) + bn_ref[...], 0.0)
    h_ref[...] = h
    hm_ref[...] = jnp.dot(h, wm_ref[...], precision=_HIGHEST,
                          preferred_element_type=jnp.float32)


def _node_embed(x, W_node, b_node, W_msg):
    return pl.pallas_call(
        _node_body,
        out_shape=(jax.ShapeDtypeStruct((N, D), jnp.float32),
                   jax.ShapeDtypeStruct((N, D), jnp.float32)),
    )(x, W_node, b_node.reshape(1, D), W_msg)


BE2 = 2000

def _edge_body(ea_ref, w1_ref, b1_ref, w2_ref, b2_ref, eme_ref, emo_ref):
    z = jnp.dot(ea_ref[...], w1_ref[...], preferred_element_type=jnp.float32)
    e = jnp.maximum(z + b1_ref[...], 0.0).astype(jnp.bfloat16)
    em = jnp.dot(e, w2_ref[...],
                 preferred_element_type=jnp.float32) + b2_ref[...]
    eme_ref[...] = em[:, :D]
    emo_ref[...] = em[:, D:]


def _edge_embed(edge_attr, W_edge, b_edge, W_msg, b_msg):
    z2 = jnp.zeros((DE, D), jnp.float32)
    w1 = jnp.block([[W_edge, z2], [z2, W_edge]]).astype(jnp.bfloat16)
    zd = jnp.zeros((D, D), jnp.float32)
    w2 = jnp.block([[W_msg, zd], [zd, W_msg]]).astype(jnp.bfloat16)
    b1 = jnp.concatenate([b_edge, b_edge]).reshape(1, 2 * D)
    b2 = jnp.concatenate([b_msg, b_msg]).reshape(1, 2 * D)
    ea2 = edge_attr.astype(jnp.bfloat16).reshape(E // 2, 2 * DE)
    return pl.pallas_call(
        _edge_body,
        grid=(E // 2 // BE2,),
        in_specs=[
            pl.BlockSpec((BE2, 2 * DE), lambda i: (i, 0)),
            pl.BlockSpec((2 * DE, 2 * D), lambda i: (0, 0)),
            pl.BlockSpec((1, 2 * D), lambda i: (0, 0)),
            pl.BlockSpec((2 * D, 2 * D), lambda i: (0, 0)),
            pl.BlockSpec((1, 2 * D), lambda i: (0, 0)),
        ],
        out_specs=(pl.BlockSpec((BE2, D), lambda i: (i, 0)),
                   pl.BlockSpec((BE2, D), lambda i: (i, 0))),
        out_shape=(jax.ShapeDtypeStruct((E // 2, D), jnp.float32),
                   jax.ShapeDtypeStruct((E // 2, D), jnp.float32)),
    )(ea2, w1, b1, w2, b2)


def _sc_body(hm_hbm, eme_hbm, emo_hbm, src_hbm, dst_hbm, acc_hbm, aux_hbm,
             src_v, dst_v, dstq_v, eme_v, emo_v, g_v, ones_v,
             acc_sh, aux_sh, sem, sem2, sem3, sem4, sem5):
    c = jax.lax.axis_index("c")
    s = jax.lax.axis_index("s")
    wid = c * NS + s

    @pl.loop(0, K)
    def _(r):
        @pl.loop(0, D, step=LANES)
        def _(j):
            g_v[r, pl.ds(j, LANES)] = jnp.zeros((LANES,), jnp.float32)
            ones_v[r, pl.ds(j, LANES)] = jnp.zeros((LANES,), jnp.float32)

    @pl.loop(0, RPS // K)
    def _(q):
        pltpu.sync_copy(g_v, acc_sh.at[pl.ds(s * RPS + q * K, K)])
    pltpu.sync_copy(g_v, aux_sh.at[pl.ds(s * QPS, QPS)])
    plsc.subcore_barrier()

    @pl.loop(0, CHUNKS)
    def _(t):
        base = wid * EPW + t * K
        cp_src = pltpu.async_copy(src_hbm.at[pl.ds(base, K)], src_v, sem2)
        cp_dst = pltpu.async_copy(dst_hbm.at[pl.ds(base, K)], dst_v, sem3)
        base2 = pl.multiple_of(wid * (EPW // 2) + t * (K // 2), 8)
        with jax.named_scope("ph_issue"):
            cp_eme = pltpu.async_copy(eme_hbm.at[pl.ds(base2, K // 2)], eme_v,
                                      sem4)
            cp_emo = pltpu.async_copy(emo_hbm.at[pl.ds(base2, K // 2)], emo_v,
                                      sem5)
            cp_src.wait()
            cp_gather = pltpu.async_copy(hm_hbm.at[src_v], g_v, sem)
            cp_dst.wait()

        with jax.named_scope("ph_ones"):
            @pl.loop(0, K, step=LANES)
            def _(r16):
                sl = pl.ds(r16, LANES)
                d16 = dst_v[sl]
                dstq_v[sl] = jax.lax.shift_right_logical(d16, 3)
                rows = jax.lax.iota(jnp.int32, LANES) + r16
                cols = (d16 & 7) * LANES
                plsc.store_scatter(ones_v, [rows, cols],
                                   jnp.full((LANES,), 1.0, jnp.float32))

        with jax.named_scope("ph_emwait"):
            cp_eme.wait()
            cp_emo.wait()
        with jax.named_scope("ph_gwait"):
            cp_gather.wait()

        with jax.named_scope("ph_compute"):
            @pl.loop(0, K // 2)
            def _(q):
                sls = [pl.ds(j, LANES) for j in range(0, D, LANES)]
                ge = [g_v[2 * q, sl] for sl in sls]
                go = [g_v[2 * q + 1, sl] for sl in sls]
                ee = [eme_v[q, sl] for sl in sls]
                eo = [emo_v[q, sl] for sl in sls]
                for sl, a, b in zip(sls, ge, ee):
                    g_v[2 * q, sl] = jnp.maximum(a + b, 0.0)
                for sl, a, b in zip(sls, go, eo):
                    g_v[2 * q + 1, sl] = jnp.maximum(a + b, 0.0)

        with jax.named_scope("ph_scat"):
            cp_acc = pltpu.async_copy(g_v, acc_sh.at[dst_v], sem2, add=True)
            pltpu.sync_copy(ones_v, aux_sh.at[dstq_v], add=True)
            cp_acc.wait()

        with jax.named_scope("ph_clear"):
            @pl.loop(0, K, step=LANES)
            def _(r16):
                sl = pl.ds(r16, LANES)
                rows = jax.lax.iota(jnp.int32, LANES) + r16
                cols = (dst_v[sl] & 7) * LANES
                plsc.store_scatter(ones_v, [rows, cols],
                                   jnp.zeros((LANES,), jnp.float32))

    plsc.subcore_barrier()

    pltpu.sync_copy(acc_sh.at[pl.ds(s * RPS, RPS)],
                    acc_hbm.at[c, pl.ds(s * RPS, RPS)])
    pltpu.sync_copy(aux_sh.at[pl.ds(s * QPS, QPS)],
                    aux_hbm.at[c, pl.ds(s * QPS, QPS)])


def _sc_aggregate(hm, eme, emo, src, dst):
    mesh = plsc.VectorSubcoreMesh(core_axis_name="c", subcore_axis_name="s")
    cp = pltpu.CompilerParams()
    if "needs_layout_passes" in pltpu.CompilerParams.__dataclass_fields__:
        cp = dataclasses.replace(cp, needs_layout_passes=False)
    kern = pl.kernel(
        _sc_body,
        compiler_params=cp,
        out_type=(jax.ShapeDtypeStruct((NC, NP, D), jnp.float32),
                  jax.ShapeDtypeStruct((NC, NQ, D), jnp.float32)),
        mesh=mesh,
        scratch_types=[
            pltpu.VMEM((K,), jnp.int32),
            pltpu.VMEM((K,), jnp.int32),
            pltpu.VMEM((K,), jnp.int32),
            pltpu.VMEM((K // 2, D), jnp.float32),
            pltpu.VMEM((K // 2, D), jnp.float32),
            pltpu.VMEM((K, D), jnp.float32),
            pltpu.VMEM((K, D), jnp.float32),
            pltpu.VMEM_SHARED((NP, D), jnp.float32),
            pltpu.VMEM_SHARED((NQ, D), jnp.float32),
            pltpu.SemaphoreType.DMA,
            pltpu.SemaphoreType.DMA,
            pltpu.SemaphoreType.DMA,
            pltpu.SemaphoreType.DMA,
            pltpu.SemaphoreType.DMA,
        ],
    )
    return kern(hm, eme, emo, src, dst)


BN = 2000

def _upd_body(h_ref, acc_ref, cnt_ref, w1_ref, w2_ref, w3_ref, bu_ref, o_ref):
    agg = acc_ref[0] + acc_ref[1]
    cnt = cnt_ref[...]
    mean = agg / jnp.maximum(cnt, 1.0)
    o = (jnp.dot(h_ref[...], w1_ref[...], precision=_HIGHEST,
                 preferred_element_type=jnp.float32)
         + jnp.dot(agg, w2_ref[...], precision=_HIGHEST,
                   preferred_element_type=jnp.float32)
         + jnp.dot(mean, w3_ref[...], precision=_HIGHEST,
                   preferred_element_type=jnp.float32)
         + bu_ref[...])
    o_ref[...] = jnp.maximum(o, 0.0)


def _node_update(h, acc, cnt, W_upd, b_upd):
    return pl.pallas_call(
        _upd_body,
        grid=(N // BN,),
        in_specs=[
            pl.BlockSpec((BN, D), lambda i: (i, 0)),
            pl.BlockSpec((2, BN, D), lambda i: (0, i, 0)),
            pl.BlockSpec((BN, 1), lambda i: (i, 0)),
            pl.BlockSpec((D, D), lambda i: (0, 0)),
            pl.BlockSpec((D, D), lambda i: (0, 0)),
            pl.BlockSpec((D, D), lambda i: (0, 0)),
            pl.BlockSpec((1, D), lambda i: (0, 0)),
        ],
        out_specs=pl.BlockSpec((BN, D), lambda i: (i, 0)),
        out_shape=jax.ShapeDtypeStruct((N, D), jnp.float32),
    )(h, acc, cnt, W_upd[0:D], W_upd[D:2 * D], W_upd[2 * D:3 * D],
      b_upd.reshape(1, D))


def kernel(x, edge_index, edge_attr, W_node, b_node, W_edge, b_edge,
           W_msg, b_msg, W_upd, b_upd):
    src = edge_index[0].astype(jnp.int32)
    dst = edge_index[1].astype(jnp.int32)
    h, hm = _node_embed(x, W_node, b_node, W_msg)
    eme, emo = _edge_embed(edge_attr, W_edge, b_edge, W_msg, b_msg)
    acc, aux = _sc_aggregate(hm, eme, emo, src, dst)
    auxs = aux[0] + aux[1]
    cnt = auxs.reshape(NQ, 8, LANES)[:, :, 0].reshape(NQ * 8, 1)[:N]
    return _node_update(h, acc, cnt, W_upd, b_upd)


# speedup vs baseline: 1.4925x; 1.0290x over previous
import dataclasses

import jax
import jax.numpy as jnp
from jax.experimental import pallas as pl
from jax.experimental.pallas import tpu as pltpu
from jax.experimental.pallas import tpu_sc as plsc

N = 10000
E = 320000
D = 128
DE = 16

NC = 2
NS = 16
LANES = 16
K = 80
EPW = E // (NC * NS)
CHUNKS = EPW // K
NP = 10240
RPS = NP // NS
NQ = 1280
QPS = NQ // NS

_HIGHEST = jax.lax.Precision.HIGHEST


def _node_body(x_ref, wn_ref, bn_ref, wm_ref, h_ref, hm_ref):
    h = jnp.maximum(
        jnp.dot(x_ref[...], wn_ref[...], precision=_HIGHEST,
                preferred_element_type=jnp.float32) + bn_ref[...], 0.0)
    h_ref[...] = h
    hm_ref[...] = jnp.dot(h, wm_ref[...], precision=_HIGHEST,
                          preferred_element_type=jnp.float32)


def _node_embed(x, W_node, b_node, W_msg):
    return pl.pallas_call(
        _node_body,
        out_shape=(jax.ShapeDtypeStruct((N, D), jnp.float32),
                   jax.ShapeDtypeStruct((N, D), jnp.float32)),
    )(x, W_node, b_node.reshape(1, D), W_msg)


BE2 = 2000

def _edge_body(ea_ref, w1_ref, b1_ref, w2_ref, b2_ref, eme_ref, emo_ref):
    z = jnp.dot(ea_ref[...], w1_ref[...], preferred_element_type=jnp.float32)
    e = jnp.maximum(z + b1_ref[...], 0.0).astype(jnp.bfloat16)
    em = jnp.dot(e, w2_ref[...],
                 preferred_element_type=jnp.float32) + b2_ref[...]
    eme_ref[...] = em[:, :D]
    emo_ref[...] = em[:, D:]


def _edge_embed(edge_attr, W_edge, b_edge, W_msg, b_msg):
    z2 = jnp.zeros((DE, D), jnp.float32)
    w1 = jnp.block([[W_edge, z2], [z2, W_edge]]).astype(jnp.bfloat16)
    zd = jnp.zeros((D, D), jnp.float32)
    w2 = jnp.block([[W_msg, zd], [zd, W_msg]]).astype(jnp.bfloat16)
    b1 = jnp.concatenate([b_edge, b_edge]).reshape(1, 2 * D)
    b2 = jnp.concatenate([b_msg, b_msg]).reshape(1, 2 * D)
    ea2 = edge_attr.astype(jnp.bfloat16).reshape(E // 2, 2 * DE)
    return pl.pallas_call(
        _edge_body,
        grid=(E // 2 // BE2,),
        in_specs=[
            pl.BlockSpec((BE2, 2 * DE), lambda i: (i, 0)),
            pl.BlockSpec((2 * DE, 2 * D), lambda i: (0, 0)),
            pl.BlockSpec((1, 2 * D), lambda i: (0, 0)),
            pl.BlockSpec((2 * D, 2 * D), lambda i: (0, 0)),
            pl.BlockSpec((1, 2 * D), lambda i: (0, 0)),
        ],
        out_specs=(pl.BlockSpec((BE2, D), lambda i: (i, 0)),
                   pl.BlockSpec((BE2, D), lambda i: (i, 0))),
        out_shape=(jax.ShapeDtypeStruct((E // 2, D), jnp.float32),
                   jax.ShapeDtypeStruct((E // 2, D), jnp.float32)),
    )(ea2, w1, b1, w2, b2)


KH = K // 2


def _sc_body(hm_hbm, eme_hbm, emo_hbm, src_hbm, dst_hbm, acc_hbm, aux_hbm,
             src_a, dst_a, eme_a, emo_a, src_b, dst_b, eme_b, emo_b,
             dstq0_v, g_v, ones_v,
             acc_sh, aux_sh, sem, sem2, sem3, sem4, sem5, sem6):
    c = jax.lax.axis_index("c")
    s = jax.lax.axis_index("s")
    wid = c * NS + s

    @pl.loop(0, K)
    def _(r):
        @pl.loop(0, D, step=LANES)
        def _(j):
            g_v[r, pl.ds(j, LANES)] = jnp.zeros((LANES,), jnp.float32)

    @pl.loop(0, LANES)
    def _(r):
        @pl.loop(0, D, step=LANES)
        def _(j):
            ones_v[r, pl.ds(j, LANES)] = jnp.zeros((LANES,), jnp.float32)

    @pl.loop(0, RPS // K)
    def _(q):
        pltpu.sync_copy(g_v, acc_sh.at[pl.ds(s * RPS + q * K, K)])
    pltpu.sync_copy(g_v, aux_sh.at[pl.ds(s * QPS, QPS)])
    plsc.subcore_barrier()

    def issue_inputs(t, src_v, dst_v, eme_v, emo_v):
        base = wid * EPW + t * K
        base2 = pl.multiple_of(wid * (EPW // 2) + t * KH, 8)
        pltpu.async_copy(src_hbm.at[pl.ds(base, K)], src_v, sem2)
        pltpu.async_copy(dst_hbm.at[pl.ds(base, K)], dst_v, sem3)
        pltpu.async_copy(eme_hbm.at[pl.ds(base2, KH)], eme_v, sem4)
        pltpu.async_copy(emo_hbm.at[pl.ds(base2, KH)], emo_v, sem5)

    def wait_idx(src_v, dst_v):
        pltpu.make_async_copy(src_hbm.at[pl.ds(0, K)], src_v, sem2).wait()
        pltpu.make_async_copy(dst_hbm.at[pl.ds(0, K)], dst_v, sem3).wait()

    def wait_em(eme_v, emo_v):
        pltpu.make_async_copy(eme_hbm.at[pl.ds(0, KH)], eme_v, sem4).wait()
        pltpu.make_async_copy(emo_hbm.at[pl.ds(0, KH)], emo_v, sem5).wait()

    def count_scatter(dst_v, grp, dstq_v):
        d16 = dst_v[pl.ds(grp * LANES, LANES)]
        dstq_v[pl.ds(0, LANES)] = jax.lax.shift_right_logical(d16, 3)
        rows = jax.lax.iota(jnp.int32, LANES)
        cols = (d16 & 7) * LANES
        plsc.store_scatter(ones_v, [rows, cols],
                           jnp.full((LANES,), 1.0, jnp.float32))
        pltpu.sync_copy(ones_v, aux_sh.at[dstq_v], add=True)
        plsc.store_scatter(ones_v, [rows, cols],
                           jnp.zeros((LANES,), jnp.float32))

    def half_step(t, cur, nxt):
        src_v, dst_v, eme_v, emo_v = cur
        wait_idx(src_v, dst_v)
        cp_gather = pltpu.async_copy(hm_hbm.at[src_v], g_v, sem)
        wait_em(eme_v, emo_v)
        tn = jnp.minimum(t + 1, CHUNKS - 1)
        issue_inputs(tn, *nxt)
        cp_gather.wait()

        @pl.loop(0, KH)
        def _(q):
            sls = [pl.ds(j, LANES) for j in range(0, D, LANES)]
            ge = [g_v[2 * q, sl] for sl in sls]
            go = [g_v[2 * q + 1, sl] for sl in sls]
            ee = [eme_v[q, sl] for sl in sls]
            eo = [emo_v[q, sl] for sl in sls]
            for sl, a, b in zip(sls, ge, ee):
                g_v[2 * q, sl] = jnp.maximum(a + b, 0.0)
            for sl, a, b in zip(sls, go, eo):
                g_v[2 * q + 1, sl] = jnp.maximum(a + b, 0.0)

        cp_acc = pltpu.async_copy(g_v, acc_sh.at[dst_v], sem6, add=True)
        for grp in range(K // LANES):
            count_scatter(dst_v, grp, dstq0_v)
        cp_acc.wait()

    buf_a = (src_a, dst_a, eme_a, emo_a)
    buf_b = (src_b, dst_b, eme_b, emo_b)
    issue_inputs(0, *buf_a)

    @pl.loop(0, CHUNKS - 1, step=2)
    def _(t):
        half_step(t, buf_a, buf_b)
        half_step(t + 1, buf_b, buf_a)

    half_step(CHUNKS - 1, buf_a, buf_b)
    wait_idx(src_b, dst_b)
    wait_em(eme_b, emo_b)

    plsc.subcore_barrier()

    pltpu.sync_copy(acc_sh.at[pl.ds(s * RPS, RPS)],
                    acc_hbm.at[c, pl.ds(s * RPS, RPS)])
    pltpu.sync_copy(aux_sh.at[pl.ds(s * QPS, QPS)],
                    aux_hbm.at[c, pl.ds(s * QPS, QPS)])


def _sc_aggregate(hm, eme, emo, src, dst):
    mesh = plsc.VectorSubcoreMesh(core_axis_name="c", subcore_axis_name="s")
    cp = pltpu.CompilerParams()
    if "needs_layout_passes" in pltpu.CompilerParams.__dataclass_fields__:
        cp = dataclasses.replace(cp, needs_layout_passes=False)
    kern = pl.kernel(
        _sc_body,
        compiler_params=cp,
        out_type=(jax.ShapeDtypeStruct((NC, NP, D), jnp.float32),
                  jax.ShapeDtypeStruct((NC, NQ, D), jnp.float32)),
        mesh=mesh,
        scratch_types=[
            pltpu.VMEM((K,), jnp.int32),
            pltpu.VMEM((K,), jnp.int32),
            pltpu.VMEM((KH, D), jnp.float32),
            pltpu.VMEM((KH, D), jnp.float32),
            pltpu.VMEM((K,), jnp.int32),
            pltpu.VMEM((K,), jnp.int32),
            pltpu.VMEM((KH, D), jnp.float32),
            pltpu.VMEM((KH, D), jnp.float32),
            pltpu.VMEM((LANES,), jnp.int32),
            pltpu.VMEM((K, D), jnp.float32),
            pltpu.VMEM((LANES, D), jnp.float32),
            pltpu.VMEM_SHARED((NP, D), jnp.float32),
            pltpu.VMEM_SHARED((NQ, D), jnp.float32),
            pltpu.SemaphoreType.DMA,
            pltpu.SemaphoreType.DMA,
            pltpu.SemaphoreType.DMA,
            pltpu.SemaphoreType.DMA,
            pltpu.SemaphoreType.DMA,
            pltpu.SemaphoreType.DMA,
        ],
    )
    return kern(hm, eme, emo, src, dst)


BN = 2000

def _upd_body(h_ref, acc_ref, cnt_ref, w1_ref, w2_ref, w3_ref, bu_ref, o_ref):
    agg = acc_ref[0] + acc_ref[1]
    cnt = cnt_ref[...]
    mean = agg / jnp.maximum(cnt, 1.0)
    o = (jnp.dot(h_ref[...], w1_ref[...], precision=_HIGHEST,
                 preferred_element_type=jnp.float32)
         + jnp.dot(agg, w2_ref[...], precision=_HIGHEST,
                   preferred_element_type=jnp.float32)
         + jnp.dot(mean, w3_ref[...], precision=_HIGHEST,
                   preferred_element_type=jnp.float32)
         + bu_ref[...])
    o_ref[...] = jnp.maximum(o, 0.0)


def _node_update(h, acc, cnt, W_upd, b_upd):
    return pl.pallas_call(
        _upd_body,
        grid=(N // BN,),
        in_specs=[
            pl.BlockSpec((BN, D), lambda i: (i, 0)),
            pl.BlockSpec((2, BN, D), lambda i: (0, i, 0)),
            pl.BlockSpec((BN, 1), lambda i: (i, 0)),
            pl.BlockSpec((D, D), lambda i: (0, 0)),
            pl.BlockSpec((D, D), lambda i: (0, 0)),
            pl.BlockSpec((D, D), lambda i: (0, 0)),
            pl.BlockSpec((1, D), lambda i: (0, 0)),
        ],
        out_specs=pl.BlockSpec((BN, D), lambda i: (i, 0)),
        out_shape=jax.ShapeDtypeStruct((N, D), jnp.float32),
    )(h, acc, cnt, W_upd[0:D], W_upd[D:2 * D], W_upd[2 * D:3 * D],
      b_upd.reshape(1, D))


def kernel(x, edge_index, edge_attr, W_node, b_node, W_edge, b_edge,
           W_msg, b_msg, W_upd, b_upd):
    src = edge_index[0].astype(jnp.int32)
    dst = edge_index[1].astype(jnp.int32)
    h, hm = _node_embed(x, W_node, b_node, W_msg)
    eme, emo = _edge_embed(edge_attr, W_edge, b_edge, W_msg, b_msg)
    acc, aux = _sc_aggregate(hm, eme, emo, src, dst)
    auxs = aux[0] + aux[1]
    cnt = auxs.reshape(NQ, 8, LANES)[:, :, 0].reshape(NQ * 8, 1)[:N]
    return _node_update(h, acc, cnt, W_upd, b_upd)
